# baseline (speedup 1.0000x reference)
import jax
import jax.numpy as jnp
from jax import lax
from jax.experimental import pallas as pl
from jax.experimental.pallas import tpu as pltpu
from jax.experimental.pallas import tpu_sc as plsc

_BATCH = 16384
_LATENT = 64
_NC = 2
_NS = 16
_NW = _NC * _NS
_BPW = _BATCH // _NW
_CHUNK = 128
_NCHUNK = _BPW // _CHUNK
_L = 16


def _mf_body(uid_hbm, iid_hbm, p_hbm, q_hbm, ub_hbm, ib_hbm, out_hbm,
             uidx_v, iidx_v, prow_v, qrow_v, ubias_v, ibias_v, sums_v,
             out_v, sem):
    wid = lax.axis_index("s") * _NC + lax.axis_index("c")

    pltpu.sync_copy(uid_hbm.at[wid], uidx_v)
    pltpu.sync_copy(iid_hbm.at[wid], iidx_v)

    copies = []
    for j in range(_NCHUNK):
        sl = pl.ds(j * _CHUNK, _CHUNK)
        copies.append(pltpu.async_copy(p_hbm.at[uidx_v.at[j]], prow_v.at[sl], sem))
        copies.append(pltpu.async_copy(q_hbm.at[iidx_v.at[j]], qrow_v.at[sl], sem))
        copies.append(pltpu.async_copy(ub_hbm.at[uidx_v.at[j]], ubias_v.at[sl], sem))
        copies.append(pltpu.async_copy(ib_hbm.at[iidx_v.at[j]], ibias_v.at[sl], sem))
    for c in copies:
        c.wait()

    def row_body(r, carry):
        acc = prow_v[r, pl.ds(0, _L)] * qrow_v[r, pl.ds(0, _L)]
        acc += prow_v[r, pl.ds(_L, _L)] * qrow_v[r, pl.ds(_L, _L)]
        acc += prow_v[r, pl.ds(2 * _L, _L)] * qrow_v[r, pl.ds(2 * _L, _L)]
        acc += prow_v[r, pl.ds(3 * _L, _L)] * qrow_v[r, pl.ds(3 * _L, _L)]
        sums_v[r] = plsc.cumsum(acc)
        return carry

    lax.fori_loop(0, _BPW, row_body, 0, unroll=8)

    last = jnp.full((_L,), _L - 1, jnp.int32)
    for g in range(_BPW // _L):
        rows = lax.iota(jnp.int32, _L) + g * _L
        dots = plsc.load_gather(sums_v, [rows, last])
        sl = pl.ds(g * _L, _L)
        out_v[sl] = dots + ubias_v[sl] + ibias_v[sl]

    pltpu.sync_copy(out_v, out_hbm.at[wid])


@jax.jit
def _mf(uid, iid, P, Q, ub, ib):
    mesh = plsc.VectorSubcoreMesh(core_axis_name="c", subcore_axis_name="s")
    f = pl.kernel(
        _mf_body,
        mesh=mesh,
        compiler_params=pltpu.CompilerParams(
            needs_layout_passes=False, use_tc_tiling_on_sc=False),
        out_type=jax.ShapeDtypeStruct((_NW, _BPW), jnp.float32),
        scratch_types=[
            pltpu.VMEM((_NCHUNK, _CHUNK), jnp.int32),
            pltpu.VMEM((_NCHUNK, _CHUNK), jnp.int32),
            pltpu.VMEM((_BPW, _LATENT), jnp.float32),
            pltpu.VMEM((_BPW, _LATENT), jnp.float32),
            pltpu.VMEM((_BPW,), jnp.float32),
            pltpu.VMEM((_BPW,), jnp.float32),
            pltpu.VMEM((_BPW, _L), jnp.float32),
            pltpu.VMEM((_BPW,), jnp.float32),
            pltpu.SemaphoreType.DMA,
        ],
    )
    return f(uid, iid, P, Q, ub, ib)


def kernel(user_id, item_id, P, Q, user_bias, item_bias):
    uid = user_id.reshape(_NW, _NCHUNK, _CHUNK)
    iid = item_id.reshape(_NW, _NCHUNK, _CHUNK)
    out = _mf(uid, iid, P, Q,
              user_bias.reshape(-1), item_bias.reshape(-1))
    return out.reshape(_BATCH)
